# baseline (speedup 1.0000x reference)
import jax
import jax.numpy as jnp
from jax.experimental import pallas as pl
from jax.experimental.pallas import tpu as pltpu

_D = 256
_H = 8
_HD = 32
_K = 32
_BQ = 256


def _proj_body(kf_ref, wk_ref, bk_ref, wv_ref, bv_ref, k_out, vaug_out):
    kf = kf_ref[...]
    k_out[...] = jax.lax.dot_general(
        kf, wk_ref[...], (((1,), (1,)), ((), ())),
        preferred_element_type=jnp.float32) + bk_ref[...]
    v = jax.lax.dot_general(
        kf, wv_ref[...], (((1,), (1,)), ((), ())),
        preferred_element_type=jnp.float32) + bv_ref[...]
    rows = v.shape[0]
    ones = jnp.ones((rows, 1), jnp.float32)
    zeros = jnp.zeros((rows, 31), jnp.float32)
    parts = []
    for h in range(_H):
        parts.append(v[:, h * _HD:(h + 1) * _HD])
        parts.append(ones)
        parts.append(zeros)
    vaug_out[...] = jnp.concatenate(parts, axis=1)


def _attn_body(qf_ref, qp_ref, kpt_ref, k_ref, vaug_ref,
               wq_ref, bq_ref, wo_ref, bo_ref, out_ref,
               dwork_ref):
    n2 = k_ref.shape[0]
    scale = _HD ** (-0.5)

    qp = qp_ref[...]
    kpt = kpt_ref[...]
    q2 = jnp.sum(qp * qp, axis=1, keepdims=True)
    k2 = jnp.sum(kpt * kpt, axis=0, keepdims=True)
    qk = jax.lax.dot_general(qp, kpt, (((1,), (0,)), ((), ())),
                             preferred_element_type=jnp.float32)
    d2 = jnp.maximum(q2 + k2 - 2.0 * qk, 0.0)
    d = jnp.sqrt(d2 + 1e-12)

    iota = jax.lax.broadcasted_iota(jnp.int32, d.shape, 1)
    big = jnp.float32(3e38)
    dwork = d
    for _i in range(_K - 1):
        jm = jnp.argmin(dwork, axis=1)[:, None]
        dwork = jnp.where(iota == jm, big, dwork)
    jm = jnp.argmin(dwork, axis=1)[:, None]
    dwork_ref[...] = jnp.where(iota == jm, big, dwork)
    bias = jnp.where(dwork_ref[...] > 1e30, 0.0, -3e38)

    q = (jax.lax.dot_general(qf_ref[...], wq_ref[...], (((1,), (1,)), ((), ())),
                             preferred_element_type=jnp.float32)
         + bq_ref[...]) * scale

    heads = []
    for h in range(_H):
        qh = q[:, h * _HD:(h + 1) * _HD]
        kh = k_ref[:, h * _HD:(h + 1) * _HD]
        vah = vaug_ref[:, h * 2 * _HD:(h + 1) * 2 * _HD]
        qn = jnp.sqrt(jnp.sum(qh * qh, axis=1, keepdims=True))
        kn2 = jnp.sum(kh * kh, axis=1, keepdims=True)
        kmax = jnp.sqrt(jnp.max(kn2))
        m_ub = qn * kmax
        s = jax.lax.dot_general(qh, kh, (((1,), (1,)), ((), ())),
                                preferred_element_type=jnp.float32)
        p = jnp.exp((s - m_ub) + bias)
        av_aug = jax.lax.dot_general(p, vah, (((1,), (0,)), ((), ())),
                                     preferred_element_type=jnp.float32)
        heads.append(av_aug[:, :_HD] / av_aug[:, _HD:_HD + 1])
    att = jnp.concatenate(heads, axis=1)

    out_ref[...] = jax.lax.dot_general(
        att, wo_ref[...], (((1,), (1,)), ((), ())),
        preferred_element_type=jnp.float32) + bo_ref[...]


def kernel(query_features, key_features, query_positions, key_positions,
           Wq, bq, Wk, bk, Wv, bv, Wo, bo):
    n1, d_model = query_features.shape
    n2 = key_features.shape[0]

    qp_pad = jnp.pad(query_positions, ((0, 0), (0, 5)))
    kpt_pad = jnp.pad(key_positions, ((0, 0), (0, 5))).T
    bq2 = bq.reshape(1, d_model)
    bk2 = bk.reshape(1, d_model)
    bv2 = bv.reshape(1, d_model)
    bo2 = bo.reshape(1, d_model)

    nblk = 8
    rows = n2 // nblk
    kv = pl.pallas_call(
        _proj_body,
        grid=(nblk,),
        in_specs=[
            pl.BlockSpec((rows, d_model), lambda i: (i, 0)),
            pl.BlockSpec((d_model, d_model), lambda i: (0, 0)),
            pl.BlockSpec((1, d_model), lambda i: (0, 0)),
            pl.BlockSpec((d_model, d_model), lambda i: (0, 0)),
            pl.BlockSpec((1, d_model), lambda i: (0, 0)),
        ],
        out_specs=[
            pl.BlockSpec((rows, d_model), lambda i: (i, 0)),
            pl.BlockSpec((rows, 2 * d_model), lambda i: (i, 0)),
        ],
        out_shape=[
            jax.ShapeDtypeStruct((n2, d_model), jnp.float32),
            jax.ShapeDtypeStruct((n2, 2 * d_model), jnp.float32),
        ],
    )(key_features, Wk, bk2, Wv, bv2)
    k_proj, v_aug = kv

    out = pl.pallas_call(
        _attn_body,
        grid=(n1 // _BQ,),
        in_specs=[
            pl.BlockSpec((_BQ, d_model), lambda i: (i, 0)),
            pl.BlockSpec((_BQ, 8), lambda i: (i, 0)),
            pl.BlockSpec((8, n2), lambda i: (0, 0)),
            pl.BlockSpec((n2, d_model), lambda i: (0, 0)),
            pl.BlockSpec((n2, 2 * d_model), lambda i: (0, 0)),
            pl.BlockSpec((d_model, d_model), lambda i: (0, 0)),
            pl.BlockSpec((1, d_model), lambda i: (0, 0)),
            pl.BlockSpec((d_model, d_model), lambda i: (0, 0)),
            pl.BlockSpec((1, d_model), lambda i: (0, 0)),
        ],
        out_specs=pl.BlockSpec((_BQ, d_model), lambda i: (i, 0)),
        out_shape=jax.ShapeDtypeStruct((n1, d_model), jnp.float32),
        scratch_shapes=[
            pltpu.VMEM((_BQ, n2), jnp.float32),
        ],
    )(query_features, qp_pad, kpt_pad, k_proj, v_aug, Wq, bq2, Wo, bo2)
    return out


# speedup vs baseline: 1.1932x; 1.1932x over previous
import jax
import jax.numpy as jnp
from jax.experimental import pallas as pl
from jax.experimental.pallas import tpu as pltpu

_D = 256
_H = 8
_HD = 32
_K = 32
_BQ = 256


def _proj_body(kf_ref, wk_ref, bk_ref, wv_ref, bv_ref, k_out, vaug_out):
    kf = kf_ref[...]
    k_out[...] = jax.lax.dot_general(
        kf, wk_ref[...], (((1,), (1,)), ((), ())),
        preferred_element_type=jnp.float32) + bk_ref[...]
    v = jax.lax.dot_general(
        kf, wv_ref[...], (((1,), (1,)), ((), ())),
        preferred_element_type=jnp.float32) + bv_ref[...]
    rows = v.shape[0]
    ones = jnp.ones((rows, 1), jnp.float32)
    zeros = jnp.zeros((rows, 31), jnp.float32)
    parts = []
    for h in range(_H):
        parts.append(v[:, h * _HD:(h + 1) * _HD])
        parts.append(ones)
        parts.append(zeros)
    vaug_out[...] = jnp.concatenate(parts, axis=1)


def _attn_body(qf_ref, qp_ref, kpt_ref, k_ref, vaug_ref,
               wq_ref, bq_ref, wo_ref, bo_ref, out_ref,
               dwork_ref):
    n2 = k_ref.shape[0]
    scale = _HD ** (-0.5)

    qp = qp_ref[...]
    kpt = kpt_ref[...]
    q2 = jnp.sum(qp * qp, axis=1, keepdims=True)
    k2 = jnp.sum(kpt * kpt, axis=0, keepdims=True)
    qk = jax.lax.dot_general(qp, kpt, (((1,), (0,)), ((), ())),
                             preferred_element_type=jnp.float32)
    d2 = jnp.maximum(q2 + k2 - 2.0 * qk, 0.0)
    d = jnp.sqrt(d2 + 1e-12)

    iota = jax.lax.broadcasted_iota(jnp.int32, d.shape, 1)
    big = jnp.float32(3e38)
    dwork_ref[...] = d

    def sel_body(_, unused):
        dwork = dwork_ref[...]
        for _i in range(7):
            jm = jnp.argmin(dwork, axis=1)[:, None]
            dwork = jnp.where(iota == jm, big, dwork)
        jm = jnp.argmin(dwork, axis=1)[:, None]
        dwork_ref[...] = jnp.where(iota == jm, big, dwork)
        return unused

    jax.lax.fori_loop(0, _K // 8, sel_body, 0)
    bias = jnp.where(dwork_ref[...] > 1e30, 0.0, -3e38)

    q = (jax.lax.dot_general(qf_ref[...], wq_ref[...], (((1,), (1,)), ((), ())),
                             preferred_element_type=jnp.float32)
         + bq_ref[...]) * scale

    heads = []
    for h in range(_H):
        qh = q[:, h * _HD:(h + 1) * _HD]
        kh = k_ref[:, h * _HD:(h + 1) * _HD]
        vah = vaug_ref[:, h * 2 * _HD:(h + 1) * 2 * _HD]
        qn = jnp.sqrt(jnp.sum(qh * qh, axis=1, keepdims=True))
        kn2 = jnp.sum(kh * kh, axis=1, keepdims=True)
        kmax = jnp.sqrt(jnp.max(kn2))
        m_ub = qn * kmax
        s = jax.lax.dot_general(qh, kh, (((1,), (1,)), ((), ())),
                                preferred_element_type=jnp.float32)
        p = jnp.exp((s - m_ub) + bias)
        av_aug = jax.lax.dot_general(p, vah, (((1,), (0,)), ((), ())),
                                     preferred_element_type=jnp.float32)
        heads.append(av_aug[:, :_HD] / av_aug[:, _HD:_HD + 1])
    att = jnp.concatenate(heads, axis=1)

    out_ref[...] = jax.lax.dot_general(
        att, wo_ref[...], (((1,), (1,)), ((), ())),
        preferred_element_type=jnp.float32) + bo_ref[...]


def kernel(query_features, key_features, query_positions, key_positions,
           Wq, bq, Wk, bk, Wv, bv, Wo, bo):
    n1, d_model = query_features.shape
    n2 = key_features.shape[0]

    qp_pad = jnp.pad(query_positions, ((0, 0), (0, 5)))
    kpt_pad = jnp.pad(key_positions, ((0, 0), (0, 5))).T
    bq2 = bq.reshape(1, d_model)
    bk2 = bk.reshape(1, d_model)
    bv2 = bv.reshape(1, d_model)
    bo2 = bo.reshape(1, d_model)

    nblk = 8
    rows = n2 // nblk
    kv = pl.pallas_call(
        _proj_body,
        grid=(nblk,),
        in_specs=[
            pl.BlockSpec((rows, d_model), lambda i: (i, 0)),
            pl.BlockSpec((d_model, d_model), lambda i: (0, 0)),
            pl.BlockSpec((1, d_model), lambda i: (0, 0)),
            pl.BlockSpec((d_model, d_model), lambda i: (0, 0)),
            pl.BlockSpec((1, d_model), lambda i: (0, 0)),
        ],
        out_specs=[
            pl.BlockSpec((rows, d_model), lambda i: (i, 0)),
            pl.BlockSpec((rows, 2 * d_model), lambda i: (i, 0)),
        ],
        out_shape=[
            jax.ShapeDtypeStruct((n2, d_model), jnp.float32),
            jax.ShapeDtypeStruct((n2, 2 * d_model), jnp.float32),
        ],
    )(key_features, Wk, bk2, Wv, bv2)
    k_proj, v_aug = kv

    out = pl.pallas_call(
        _attn_body,
        grid=(n1 // _BQ,),
        in_specs=[
            pl.BlockSpec((_BQ, d_model), lambda i: (i, 0)),
            pl.BlockSpec((_BQ, 8), lambda i: (i, 0)),
            pl.BlockSpec((8, n2), lambda i: (0, 0)),
            pl.BlockSpec((n2, d_model), lambda i: (0, 0)),
            pl.BlockSpec((n2, 2 * d_model), lambda i: (0, 0)),
            pl.BlockSpec((d_model, d_model), lambda i: (0, 0)),
            pl.BlockSpec((1, d_model), lambda i: (0, 0)),
            pl.BlockSpec((d_model, d_model), lambda i: (0, 0)),
            pl.BlockSpec((1, d_model), lambda i: (0, 0)),
        ],
        out_specs=pl.BlockSpec((_BQ, d_model), lambda i: (i, 0)),
        out_shape=jax.ShapeDtypeStruct((n1, d_model), jnp.float32),
        scratch_shapes=[
            pltpu.VMEM((_BQ, n2), jnp.float32),
        ],
    )(query_features, qp_pad, kpt_pad, k_proj, v_aug, Wq, bq2, Wo, bo2)
    return out
